# baseline (speedup 1.0000x reference)
import functools

import jax
import jax.numpy as jnp
from jax import lax
from jax.experimental import pallas as pl
from jax.experimental.pallas import tpu as pltpu
from jax.experimental.pallas import tpu_sc as plsc

_EPS = 1e-5
_DROP_RATIO = 0.25


def _score_kernel(x_ref, lnw_ref, lnb_ref, w_ref, b_ref, out_ref):
    xb = x_ref[0]
    c = xb.shape[1]
    lnw = lnw_ref[...]
    lnb = lnb_ref[...]
    wr = w_ref[...]
    wp = lnw * wr
    ones = jnp.ones_like(wp)
    w2 = jnp.concatenate([wp, ones], axis=0)
    y = lax.dot_general(xb, w2, (((1,), (1,)), ((), ())),
                        preferred_element_type=jnp.float32)
    s2 = lax.dot_general(xb * xb, ones, (((1,), (1,)), ((), ())),
                         preferred_element_type=jnp.float32)
    dot = y[:, 0:1]
    mean = y[:, 1:2] * (1.0 / c)
    var = s2 * (1.0 / c) - mean * mean
    inv = lax.rsqrt(var + _EPS)
    sum_wp = jnp.sum(wp, axis=1, keepdims=True)
    off = jnp.sum(lnb * wr, axis=1, keepdims=True) + b_ref[...]
    score = (dot - mean * sum_wp) * inv + off
    out_ref[...] = score[:, 0][None, None, :]


def _tc_scores(x, ln_w, ln_b, w, b):
    B, N, C = x.shape
    BN = 2048
    NB = N // BN
    lnw2 = ln_w.reshape(1, C)
    lnb2 = ln_b.reshape(1, C)
    w2 = w.reshape(1, C)
    b2 = b.reshape(1, 1)
    scores = pl.pallas_call(
        _score_kernel,
        grid=(B, NB),
        in_specs=[
            pl.BlockSpec((1, BN, C), lambda i, j: (i, j, 0)),
            pl.BlockSpec((1, C), lambda i, j: (0, 0)),
            pl.BlockSpec((1, C), lambda i, j: (0, 0)),
            pl.BlockSpec((1, C), lambda i, j: (0, 0)),
            pl.BlockSpec((1, 1), lambda i, j: (0, 0)),
        ],
        out_specs=pl.BlockSpec((1, 1, BN), lambda i, j, nb=NB: (i * nb + j, 0, 0)),
        out_shape=jax.ShapeDtypeStruct((B * NB, 1, BN), jnp.float32),
    )(x, lnw2, lnb2, w2, b2)
    return scores.reshape(B, N)


def _sc_mask_kernel(s_hbm, o_hbm, srow, krow, mrow, *, n, k):
    nchunks = n // 16
    wid = lax.axis_index("s") * 2 + lax.axis_index("c")
    b = s_hbm.shape[0]

    @pl.when(wid < b)
    def _():
        pltpu.sync_copy(s_hbm.at[wid], srow)
        onev = jnp.ones((16,), jnp.int32)
        zerov = jnp.zeros((16,), jnp.int32)
        lanes = lax.iota(jnp.int32, 16)
        dn = lax.GatherDimensionNumbers(
            offset_dims=(), collapsed_slice_dims=(0,), start_index_map=(0,))

        def gat(v, idx):
            return lax.gather(v, idx.reshape(16, 1), dn, (1,),
                              mode=lax.GatherScatterMode.PROMISE_IN_BOUNDS)

        def rot_reduce(v):
            for m in (8, 4, 2, 1):
                v = v + gat(v, jnp.bitwise_and(lanes + m, 15))
            return v

        def avg(a, c):
            return (a & c) + ((a ^ c) >> 1)

        def count3_le(m1, m2, m3):
            def body(j, accs):
                a1, a2, a3 = accs
                kv = krow[pl.ds(j * 16, 16)]
                a1 = a1 + jnp.where(kv <= m1, onev, zerov)
                a2 = a2 + jnp.where(kv <= m2, onev, zerov)
                a3 = a3 + jnp.where(kv <= m3, onev, zerov)
                return (a1, a2, a3)

            a1, a2, a3 = lax.fori_loop(0, nchunks, body,
                                       (zerov, zerov, zerov), unroll=8)
            return rot_reduce(a1), rot_reduce(a2), rot_reduce(a3)

        def key_body(j, carry):
            v = srow[pl.ds(j * 16, 16)]
            i = lax.bitcast_convert_type(v, jnp.int32)
            krow[pl.ds(j * 16, 16)] = jnp.where(
                i >= 0, i, i ^ jnp.int32(0x7FFFFFFF))
            return carry

        lax.fori_loop(0, nchunks, key_body, 0, unroll=4)

        kv_tgt = jnp.full((16,), k, jnp.int32)

        def bisect_body(_s, state):
            lo, hi = state
            m2 = avg(lo, hi)
            m1 = avg(lo, m2)
            m3 = avg(m2 + onev, hi)
            c1, c2, c3 = count3_le(m1, m2, m3)
            p1 = c1 >= kv_tgt
            p2 = c2 >= kv_tgt
            p3 = c3 >= kv_tgt
            hi = jnp.where(p1, m1, jnp.where(p2, m2, jnp.where(p3, m3, hi)))
            lo = jnp.where(p1, lo,
                           jnp.where(p2, m1 + onev,
                                     jnp.where(p3, m2 + onev, m3 + onev)))
            return (lo, hi)

        lo, hi = lax.fori_loop(
            0, 16, bisect_body,
            (jnp.full((16,), -2147483648, jnp.int32),
             jnp.full((16,), 2147483647, jnp.int32)))
        t = lo

        def less_body(j, acc):
            kv = krow[pl.ds(j * 16, 16)]
            return acc + jnp.where(kv < t, onev, zerov)

        c_less = rot_reduce(
            lax.fori_loop(0, nchunks, less_body, zerov, unroll=8))
        r = kv_tgt - c_less

        ninf = jnp.full((16,), -jnp.inf, jnp.float32)
        zf = jnp.zeros((16,), jnp.float32)
        lane15 = jnp.full((16,), 15, jnp.int32)

        def mask_body(j, cnt_eq):
            kv = krow[pl.ds(j * 16, 16)]
            e = kv == t
            ei = jnp.where(e, onev, zerov)
            incl = ei
            for m in (1, 2, 4, 8):
                sh = gat(incl, jnp.bitwise_and(lanes - m, 15))
                incl = incl + jnp.where(lanes >= m, sh, zerov)
            excl = incl - ei
            sel = (kv < t) | (e & ((cnt_eq + excl) < r))
            mrow[pl.ds(j * 16, 16)] = jnp.where(sel, ninf, zf)
            return cnt_eq + gat(incl, lane15)

        lax.fori_loop(0, nchunks, mask_body, zerov, unroll=4)
        pltpu.sync_copy(mrow, o_hbm.at[wid])


def _sc_mask(scores):
    B, N = scores.shape
    k = int(round(N * _DROP_RATIO))
    mesh = plsc.VectorSubcoreMesh(core_axis_name="c", subcore_axis_name="s")
    f = functools.partial(
        pl.kernel,
        mesh=mesh,
        out_type=jax.ShapeDtypeStruct((B, N), jnp.float32),
        scratch_types=[
            pltpu.VMEM((N,), jnp.float32),
            pltpu.VMEM((N,), jnp.int32),
            pltpu.VMEM((N,), jnp.float32),
        ],
    )(functools.partial(_sc_mask_kernel, n=N, k=k))
    return f(scores)


def kernel(x, ln_w, ln_b, w, b):
    scores = _tc_scores(x, ln_w, ln_b, w, b)
    mask = _sc_mask(scores)
    return mask[..., None]


# speedup vs baseline: 1.1052x; 1.1052x over previous
import functools

import jax
import jax.numpy as jnp
from jax import lax
from jax.experimental import pallas as pl
from jax.experimental.pallas import tpu as pltpu
from jax.experimental.pallas import tpu_sc as plsc

_EPS = 1e-5
_DROP_RATIO = 0.25


def _score_kernel(x_ref, lnw_ref, lnb_ref, w_ref, b_ref, out_ref):
    xb = x_ref[0]
    c = xb.shape[1]
    lnw = lnw_ref[...]
    lnb = lnb_ref[...]
    wr = w_ref[...]
    wp = lnw * wr
    ones = jnp.ones_like(wp)
    w2 = jnp.concatenate([wp, ones], axis=0)
    y = lax.dot_general(xb, w2, (((1,), (1,)), ((), ())),
                        preferred_element_type=jnp.float32)
    s2 = lax.dot_general(xb * xb, ones, (((1,), (1,)), ((), ())),
                         preferred_element_type=jnp.float32)
    dot = y[:, 0:1]
    mean = y[:, 1:2] * (1.0 / c)
    var = s2 * (1.0 / c) - mean * mean
    inv = lax.rsqrt(var + _EPS)
    sum_wp = jnp.sum(wp, axis=1, keepdims=True)
    off = jnp.sum(lnb * wr, axis=1, keepdims=True) + b_ref[...]
    score = (dot - mean * sum_wp) * inv + off
    out_ref[...] = score[:, 0][None, None, :]


def _tc_scores(x, ln_w, ln_b, w, b):
    B, N, C = x.shape
    BN = 2048
    NB = N // BN
    lnw2 = ln_w.reshape(1, C)
    lnb2 = ln_b.reshape(1, C)
    w2 = w.reshape(1, C)
    b2 = b.reshape(1, 1)
    scores = pl.pallas_call(
        _score_kernel,
        grid=(B, NB),
        in_specs=[
            pl.BlockSpec((1, BN, C), lambda i, j: (i, j, 0)),
            pl.BlockSpec((1, C), lambda i, j: (0, 0)),
            pl.BlockSpec((1, C), lambda i, j: (0, 0)),
            pl.BlockSpec((1, C), lambda i, j: (0, 0)),
            pl.BlockSpec((1, 1), lambda i, j: (0, 0)),
        ],
        out_specs=pl.BlockSpec((1, 1, BN), lambda i, j, nb=NB: (i * nb + j, 0, 0)),
        out_shape=jax.ShapeDtypeStruct((B * NB, 1, BN), jnp.float32),
    )(x, lnw2, lnb2, w2, b2)
    return scores.reshape(B, N)


def _sc_mask_kernel(s_hbm, o_hbm, srow, krow, mrow, *, n, k):
    nchunks = n // 16
    wid = lax.axis_index("s") * 2 + lax.axis_index("c")
    b = s_hbm.shape[0]

    @pl.when(wid < b)
    def _():
        pltpu.sync_copy(s_hbm.at[wid], srow)
        onev = jnp.ones((16,), jnp.int32)
        zerov = jnp.zeros((16,), jnp.int32)
        lanes = lax.iota(jnp.int32, 16)
        dn = lax.GatherDimensionNumbers(
            offset_dims=(), collapsed_slice_dims=(0,), start_index_map=(0,))

        def gat(v, idx):
            return lax.gather(v, idx.reshape(16, 1), dn, (1,),
                              mode=lax.GatherScatterMode.PROMISE_IN_BOUNDS)

        def rot_reduce(v):
            for m in (8, 4, 2, 1):
                v = v + gat(v, jnp.bitwise_and(lanes + m, 15))
            return v

        def avg(a, c):
            return (a & c) + ((a ^ c) >> 1)

        def count3_le(m1, m2, m3):
            def body(j, accs):
                a1, a2, a3 = accs
                kv = krow[pl.ds(j * 16, 16)]
                a1 = a1 + jnp.where(kv <= m1, onev, zerov)
                a2 = a2 + jnp.where(kv <= m2, onev, zerov)
                a3 = a3 + jnp.where(kv <= m3, onev, zerov)
                return (a1, a2, a3)

            a1, a2, a3 = lax.fori_loop(0, nchunks, body,
                                       (zerov, zerov, zerov), unroll=4)
            return rot_reduce(a1), rot_reduce(a2), rot_reduce(a3)

        def key_body(j, carry):
            v = srow[pl.ds(j * 16, 16)]
            i = lax.bitcast_convert_type(v, jnp.int32)
            krow[pl.ds(j * 16, 16)] = jnp.where(
                i >= 0, i, i ^ jnp.int32(0x7FFFFFFF))
            return carry

        lax.fori_loop(0, nchunks, key_body, 0, unroll=4)

        kv_tgt = jnp.full((16,), k, jnp.int32)

        def bisect_body(_s, state):
            lo, hi = state
            m2 = avg(lo, hi)
            m1 = avg(lo, m2)
            m3 = avg(m2 + onev, hi)
            c1, c2, c3 = count3_le(m1, m2, m3)
            p1 = c1 >= kv_tgt
            p2 = c2 >= kv_tgt
            p3 = c3 >= kv_tgt
            hi = jnp.where(p1, m1, jnp.where(p2, m2, jnp.where(p3, m3, hi)))
            lo = jnp.where(p1, lo,
                           jnp.where(p2, m1 + onev,
                                     jnp.where(p3, m2 + onev, m3 + onev)))
            return (lo, hi)

        lo, hi = lax.fori_loop(
            0, 16, bisect_body,
            (jnp.full((16,), -2147483648, jnp.int32),
             jnp.full((16,), 2147483647, jnp.int32)))
        t = lo

        def less_body(j, acc):
            kv = krow[pl.ds(j * 16, 16)]
            return acc + jnp.where(kv < t, onev, zerov)

        c_less = rot_reduce(
            lax.fori_loop(0, nchunks, less_body, zerov, unroll=8))
        r = kv_tgt - c_less

        ninf = jnp.full((16,), -jnp.inf, jnp.float32)
        zf = jnp.zeros((16,), jnp.float32)
        lane15 = jnp.full((16,), 15, jnp.int32)

        def mask_body(j, cnt_eq):
            kv = krow[pl.ds(j * 16, 16)]
            e = kv == t
            ei = jnp.where(e, onev, zerov)
            incl = ei
            for m in (1, 2, 4, 8):
                sh = gat(incl, jnp.bitwise_and(lanes - m, 15))
                incl = incl + jnp.where(lanes >= m, sh, zerov)
            excl = incl - ei
            sel = (kv < t) | (e & ((cnt_eq + excl) < r))
            mrow[pl.ds(j * 16, 16)] = jnp.where(sel, ninf, zf)
            return cnt_eq + gat(incl, lane15)

        lax.fori_loop(0, nchunks, mask_body, zerov, unroll=4)
        pltpu.sync_copy(mrow, o_hbm.at[wid])


def _sc_mask(scores):
    B, N = scores.shape
    k = int(round(N * _DROP_RATIO))
    mesh = plsc.VectorSubcoreMesh(core_axis_name="c", subcore_axis_name="s")
    f = functools.partial(
        pl.kernel,
        mesh=mesh,
        out_type=jax.ShapeDtypeStruct((B, N), jnp.float32),
        scratch_types=[
            pltpu.VMEM((N,), jnp.float32),
            pltpu.VMEM((N,), jnp.int32),
            pltpu.VMEM((N,), jnp.float32),
        ],
    )(functools.partial(_sc_mask_kernel, n=N, k=k))
    return f(scores)


def kernel(x, ln_w, ln_b, w, b):
    scores = _tc_scores(x, ln_w, ln_b, w, b)
    mask = _sc_mask(scores)
    return mask[..., None]
